# baseline (speedup 1.0000x reference)
import functools

import jax
import jax.numpy as jnp
from jax import lax
from jax.experimental import pallas as pl
from jax.experimental.pallas import tpu as pltpu
from jax.experimental.pallas import tpu_sc as plsc

N = 10000
N_PAD = 10240
TRASH = N
E_TOT = 160000 + N
E_PAD = 172032
CHUNK = 128
ROWS_PER_TEC = N_PAD // 16
F32 = jnp.float32

_mesh = lambda: plsc.VectorSubcoreMesh(core_axis_name="c", subcore_axis_name="s")


NG = 8


def _split_cols(hb):
    return tuple(hb[:, g * 64:(g + 1) * 64] for g in range(NG))


def _tc1_body(x_ref, w_ref, p_ref, *outs):
    hb = jnp.dot(x_ref[...], w_ref[...], preferred_element_type=F32)
    ab = jnp.dot(hb, p_ref[...], preferred_element_type=F32)
    hs = _split_cols(hb)
    for g in range(NG):
        outs[g][...] = hs[g]
    z = jnp.zeros((hb.shape[0], 8), F32)
    outs[NG][...] = jnp.concatenate([ab[:, 0:8], z], axis=1)
    outs[NG + 1][...] = jnp.concatenate([ab[:, 8:16], z], axis=1)


def _tc1(x_pad, W1, P1, bn=256):
    grid = (N_PAD // bn,)
    fs = jax.ShapeDtypeStruct
    return pl.pallas_call(
        _tc1_body,
        grid=grid,
        in_specs=[pl.BlockSpec((bn, 256), lambda i: (i, 0)),
                  pl.BlockSpec((256, 512), lambda i: (0, 0)),
                  pl.BlockSpec((512, 16), lambda i: (0, 0))],
        out_specs=[pl.BlockSpec((bn, 64), lambda i: (i, 0))] * NG
                  + [pl.BlockSpec((bn, 16), lambda i: (i, 0))] * 2,
        out_shape=[fs((N_PAD, 64), F32)] * NG + [fs((N_PAD, 16), F32)] * 2,
    )(x_pad, W1, P1)


def _tc_mid_body(*refs):
    og = refs[0:NG]
    den, b_ref, w_ref, p_ref = refs[NG:NG + 4]
    outs = refs[NG + 4:]
    bn = og[0].shape[0]
    d = den[...]
    cols = [og[g][...] / (d[:, g:g + 1] + 1e-16) for g in range(NG)]
    sb = jnp.concatenate(cols, axis=1) + b_ref[...]
    sb = jnp.where(sb > 0, sb, jnp.exp(sb) - 1.0)
    hb = jnp.dot(sb, w_ref[...], preferred_element_type=F32)
    ab = jnp.dot(hb, p_ref[...], preferred_element_type=F32)
    if len(outs) == NG + 2:
        hs = _split_cols(hb)
        for g in range(NG):
            outs[g][...] = hs[g]
        z = jnp.zeros((bn, 8), F32)
        outs[NG][...] = jnp.concatenate([ab[:, 0:8], z], axis=1)
        outs[NG + 1][...] = jnp.concatenate([ab[:, 8:16], z], axis=1)
    else:
        h3_, ast, adt = outs
        h3_[...] = hb
        z15 = jnp.zeros((bn, 15), F32)
        ast[...] = jnp.concatenate([ab[:, 0:1], z15], axis=1)
        adt[...] = jnp.concatenate([ab[:, 8:9], z15], axis=1)


def _tc_mid(og, den, b, W, P, n_out, bn=256):
    grid = (N_PAD // bn,)
    fs = jax.ShapeDtypeStruct
    m = W.shape[1]
    if n_out == 512:
        out_specs = ([pl.BlockSpec((bn, 64), lambda i: (i, 0))] * NG
                     + [pl.BlockSpec((bn, 16), lambda i: (i, 0))] * 2)
        out_shape = [fs((N_PAD, 64), F32)] * NG + [fs((N_PAD, 16), F32)] * 2
    else:
        out_specs = ([pl.BlockSpec((bn, 64), lambda i: (i, 0))]
                     + [pl.BlockSpec((bn, 16), lambda i: (i, 0))] * 2)
        out_shape = [fs((N_PAD, 64), F32)] + [fs((N_PAD, 16), F32)] * 2
    return pl.pallas_call(
        _tc_mid_body,
        grid=grid,
        in_specs=[pl.BlockSpec((bn, 64), lambda i: (i, 0))] * NG
                 + [pl.BlockSpec((bn, 16), lambda i: (i, 0)),
                    pl.BlockSpec((1, 512), lambda i: (0, 0)),
                    pl.BlockSpec((512, m), lambda i: (0, 0)),
                    pl.BlockSpec((m, 16), lambda i: (0, 0))],
        out_specs=out_specs,
        out_shape=out_shape,
    )(*og, den, b, W, P)


def _tc_final_body(p0, p1, d0, d1, b_ref, out_ref):
    den = d0[...][:, 0:1] + d1[...][:, 0:1] + 1e-16
    out_ref[...] = (p0[...] + p1[...]) / den + b_ref[...]


def _tc_final(p0, p1, d0, d1, b3, bn=400):
    grid = (N // bn,)
    return pl.pallas_call(
        _tc_final_body,
        grid=grid,
        in_specs=[pl.BlockSpec((bn, 64), lambda i: (i, 0))] * 2
                 + [pl.BlockSpec((bn, 16), lambda i: (i, 0))] * 2
                 + [pl.BlockSpec((1, 64), lambda i: (0, 0))],
        out_specs=pl.BlockSpec((bn, 64), lambda i: (i, 0)),
        out_shape=jax.ShapeDtypeStruct((N, 64), F32),
    )(p0, p1, d0, d1, b3)


_ZERO16 = None


def _zero_rows(buf, width):
    z = jnp.zeros((16,), F32)

    def body(r, _):
        for j in range(width // 16):
            buf[r, pl.ds(j * 16, 16)] = z
        return 0

    lax.fori_loop(0, CHUNK, body, 0, unroll=4)


def _zero_shared(acc, buf, s, width):
    for k in range(ROWS_PER_TEC // CHUNK):
        pltpu.sync_copy(buf, acc.at[pl.ds(s * ROWS_PER_TEC + k * CHUNK, CHUNK)])


def _alpha_chunk(sbuf, dbuf, abuf):
    def alpha_body(e, _):
        av = sbuf[e, :] + dbuf[e, :]
        av = jnp.maximum(av, 0.2 * av)
        abuf[e, :] = jnp.exp(av)
        return 0

    lax.fori_loop(0, CHUNK, alpha_body, 0, unroll=8)


def _edge_pass(s_row0, n_chunks, sidx_all, didx_all, hg,
               sbuf, dbuf, abuf, hbuf, acc, dacc,
               sa, sd, sh, ssc, ssd, sal,
               a_col, width, add_denom,
               ast_hbm=None, adt_hbm=None,
               alpha_out=None, alpha_in=None, row_base=None):
    nvec = width // 16
    compute_alpha = alpha_in is None

    def issue_gathers(ch, p):
        if compute_alpha:
            pltpu.async_copy(ast_hbm.at[sidx_all.at[s_row0 + ch]], sbuf[p], sa[p])
            pltpu.async_copy(adt_hbm.at[didx_all.at[s_row0 + ch]], dbuf[p], sd[p])
        else:
            pltpu.async_copy(alpha_in.at[pl.ds(row_base + ch * CHUNK, CHUNK)],
                             abuf[p], sa[p])
        pltpu.async_copy(hg.at[sidx_all.at[s_row0 + ch]], hbuf[p], sh[p])

    def wait_scatters(p):
        pltpu.make_async_copy(hbuf[p], acc.at[pl.ds(0, CHUNK)], ssc[p]).wait()
        if add_denom:
            pltpu.make_async_copy(abuf[p], dacc.at[pl.ds(0, CHUNK)], ssd[p]).wait()
        if alpha_out is not None:
            pltpu.make_async_copy(abuf[p], alpha_out.at[pl.ds(0, CHUNK)],
                                  sal[p]).wait()

    issue_gathers(0, 0)

    def loop_body(tt, _):
        for b in (0, 1, 2):
            ch = 3 * tt + b
            p, q = b, (b + 1) % 3

            @pl.when(ch >= 2)
            def _():
                wait_scatters(q)

            @pl.when(ch + 1 < n_chunks)
            def _():
                issue_gathers(ch + 1, q)

            if compute_alpha:
                pltpu.make_async_copy(ast_hbm.at[pl.ds(0, CHUNK)], sbuf[p],
                                      sa[p]).wait()
                pltpu.make_async_copy(adt_hbm.at[pl.ds(0, CHUNK)], dbuf[p],
                                      sd[p]).wait()
                _alpha_chunk(sbuf[p], dbuf[p], abuf[p])
                if alpha_out is not None:
                    pltpu.async_copy(abuf[p],
                                     alpha_out.at[pl.ds(row_base + ch * CHUNK,
                                                        CHUNK)], sal[p])
                if add_denom:
                    pltpu.async_copy(abuf[p], dacc.at[didx_all.at[s_row0 + ch]],
                                     ssd[p], add=True)
            else:
                pltpu.make_async_copy(alpha_in.at[pl.ds(0, CHUNK)], abuf[p],
                                      sa[p]).wait()
            pltpu.make_async_copy(hg.at[pl.ds(0, CHUNK)], hbuf[p], sh[p]).wait()

            def scale_body(e, _):
                av = abuf[p][e, :]
                for j in range(nvec):
                    aa = av[a_col + (j * 16) // 64]
                    hbuf[p][e, pl.ds(j * 16, 16)] = hbuf[p][e, pl.ds(j * 16, 16)] * aa
                return 0

            lax.fori_loop(0, CHUNK, scale_body, 0, unroll=8)
            pltpu.async_copy(hbuf[p], acc.at[didx_all.at[s_row0 + ch]],
                             ssc[p], add=True)
        return 0

    lax.fori_loop(0, n_chunks // 3, loop_body, 0)
    wait_scatters(1)
    wait_scatters(2)


def _sc12_build():
    fs = jax.ShapeDtypeStruct
    per_tec = E_PAD // 16
    n_chunks = per_tec // CHUNK

    @functools.partial(
        pl.kernel,
        out_type=[fs((N_PAD, 64), F32)] * NG + [fs((N_PAD, 16), F32),
                                                fs((2 * E_PAD, 16), F32)],
        mesh=_mesh(),
        compiler_params=pltpu.CompilerParams(use_tc_tiling_on_sc=False),
        scratch_types=[
            pltpu.VMEM((n_chunks, CHUNK), jnp.int32),
            pltpu.VMEM((n_chunks, CHUNK), jnp.int32),
            [pltpu.VMEM((CHUNK, 16), F32)] * 3,
            [pltpu.VMEM((CHUNK, 16), F32)] * 3,
            [pltpu.VMEM((CHUNK, 16), F32)] * 3,
            [pltpu.VMEM((CHUNK, 64), F32)] * 3,
            pltpu.VMEM_SHARED((N_PAD, 64), F32),
            pltpu.VMEM_SHARED((N_PAD, 16), F32),
            [pltpu.SemaphoreType.DMA] * 3,
            [pltpu.SemaphoreType.DMA] * 3,
            [pltpu.SemaphoreType.DMA] * 3,
            [pltpu.SemaphoreType.DMA] * 3,
            [pltpu.SemaphoreType.DMA] * 3,
            [pltpu.SemaphoreType.DMA] * 3,
        ],
    )
    def sc12(src_hbm, dst_hbm, ast_hbm, adt_hbm, *rest):
        hgs = rest[0:NG]
        ogs = rest[NG:2 * NG]
        den_out = rest[2 * NG]
        alpha_hbm = rest[2 * NG + 1]
        (sidx_all, didx_all, sbuf, dbuf, abuf, hbuf, acc, dacc,
         sa, sd, sh, ssc, ssd, sal) = rest[2 * NG + 2:]
        c = lax.axis_index("c")
        s = lax.axis_index("s")

        pltpu.sync_copy(src_hbm.at[pl.ds(s * n_chunks, n_chunks)], sidx_all)
        pltpu.sync_copy(dst_hbm.at[pl.ds(s * n_chunks, n_chunks)], didx_all)

        _zero_rows(hbuf[0], 64)
        _zero_shared(acc, hbuf[0], s, 64)
        _zero_rows(sbuf[0], 16)
        _zero_shared(dacc, sbuf[0], s, 16)
        plsc.subcore_barrier()

        def dump(og):
            pltpu.sync_copy(acc.at[pl.ds(s * ROWS_PER_TEC, ROWS_PER_TEC)],
                            og.at[pl.ds(s * ROWS_PER_TEC, ROWS_PER_TEC)])

        def do_groups(c_val):
            row_base = c_val * E_PAD + s * per_tec
            for k in range(4):
                g = 4 * c_val + k
                add_denom = (c_val == 0 and k == 0)
                if k > 0:
                    _zero_rows(hbuf[0], 64)
                    _zero_shared(acc, hbuf[0], s, 64)
                    plsc.subcore_barrier()
                if k == 0:
                    _edge_pass(0, n_chunks, sidx_all, didx_all,
                               hgs[g], sbuf, dbuf, abuf, hbuf, acc, dacc,
                               sa, sd, sh, ssc, ssd, sal, g, 64, add_denom,
                               ast_hbm=ast_hbm, adt_hbm=adt_hbm,
                               alpha_out=alpha_hbm, row_base=row_base)
                else:
                    _edge_pass(0, n_chunks, sidx_all, didx_all,
                               hgs[g], sbuf, dbuf, abuf, hbuf, acc, dacc,
                               sa, sd, sh, ssc, ssd, sal, g, 64, False,
                               alpha_in=alpha_hbm, row_base=row_base)
                plsc.subcore_barrier()
                dump(ogs[g])
                if add_denom:
                    pltpu.sync_copy(
                        dacc.at[pl.ds(s * ROWS_PER_TEC, ROWS_PER_TEC)],
                        den_out.at[pl.ds(s * ROWS_PER_TEC, ROWS_PER_TEC)])

        @pl.when(c == 0)
        def _():
            do_groups(0)

        @pl.when(c == 1)
        def _():
            do_groups(1)

    return sc12


def _sc3_build():
    fs = jax.ShapeDtypeStruct
    per_tec = E_PAD // 32

    n_chunks = per_tec // CHUNK

    @functools.partial(
        pl.kernel,
        out_type=[fs((2, N_PAD, 64), F32), fs((2, N_PAD, 16), F32)],
        mesh=_mesh(),
        compiler_params=pltpu.CompilerParams(use_tc_tiling_on_sc=False),
        scratch_types=[
            pltpu.VMEM((n_chunks, CHUNK), jnp.int32),
            pltpu.VMEM((n_chunks, CHUNK), jnp.int32),
            [pltpu.VMEM((CHUNK, 16), F32)] * 3,
            [pltpu.VMEM((CHUNK, 16), F32)] * 3,
            [pltpu.VMEM((CHUNK, 16), F32)] * 3,
            [pltpu.VMEM((CHUNK, 64), F32)] * 3,
            pltpu.VMEM_SHARED((N_PAD, 64), F32),
            pltpu.VMEM_SHARED((N_PAD, 16), F32),
            [pltpu.SemaphoreType.DMA] * 3,
            [pltpu.SemaphoreType.DMA] * 3,
            [pltpu.SemaphoreType.DMA] * 3,
            [pltpu.SemaphoreType.DMA] * 3,
            [pltpu.SemaphoreType.DMA] * 3,
        ],
    )
    def sc3(src_hbm, dst_hbm, ast_hbm, adt_hbm, h3_hbm,
            op, dp,
            sidx_all, didx_all, sbuf, dbuf, abuf, hbuf, acc, dacc,
            sa, sd, sh, ssc, ssd):
        c = lax.axis_index("c")
        s = lax.axis_index("s")
        wid = c * 16 + s

        pltpu.sync_copy(src_hbm.at[pl.ds(wid * n_chunks, n_chunks)], sidx_all)
        pltpu.sync_copy(dst_hbm.at[pl.ds(wid * n_chunks, n_chunks)], didx_all)

        _zero_rows(hbuf[0], 64)
        _zero_shared(acc, hbuf[0], s, 64)
        _zero_rows(sbuf[0], 16)
        _zero_shared(dacc, sbuf[0], s, 16)
        plsc.subcore_barrier()

        _edge_pass(0, n_chunks, sidx_all, didx_all, h3_hbm,
                   sbuf, dbuf, abuf, hbuf, acc, dacc,
                   sa, sd, sh, ssc, ssd, None, 0, 64, True,
                   ast_hbm=ast_hbm, adt_hbm=adt_hbm)
        plsc.subcore_barrier()
        pltpu.sync_copy(acc.at[pl.ds(s * ROWS_PER_TEC, ROWS_PER_TEC)],
                        op.at[c, pl.ds(s * ROWS_PER_TEC, ROWS_PER_TEC)])
        pltpu.sync_copy(dacc.at[pl.ds(s * ROWS_PER_TEC, ROWS_PER_TEC)],
                        dp.at[c, pl.ds(s * ROWS_PER_TEC, ROWS_PER_TEC)])

    return sc3


def _bp(att):
    H, C = att.shape
    eye = jnp.eye(H, dtype=att.dtype)
    return (att[:, :, None] * eye[:, None, :]).reshape(H * C, H)


def _build_p(att_s, att_d):
    H = att_s.shape[0]
    ps, pd = _bp(att_s), _bp(att_d)
    z = jnp.zeros((ps.shape[0], 8 - H), att_s.dtype)
    return jnp.concatenate([ps, z, pd, z], axis=1)


def kernel(x, edge_index, W1, att_src1, att_dst1, b1,
           W2, att_src2, att_dst2, b2,
           W3, att_src3, att_dst3, b3):
    ei = edge_index.astype(jnp.int32)
    loop = jnp.arange(N, dtype=jnp.int32)
    pad_n = E_PAD - E_TOT
    src = jnp.concatenate([ei[0], loop, jnp.zeros((pad_n,), jnp.int32)])
    dst = jnp.concatenate([ei[1], loop, jnp.full((pad_n,), TRASH, jnp.int32)])
    src = src.reshape(E_PAD // CHUNK, CHUNK)
    dst = dst.reshape(E_PAD // CHUNK, CHUNK)
    x_pad = jnp.pad(x, ((0, N_PAD - N), (0, 0)))

    P1 = _build_p(att_src1, att_dst1)
    P2 = _build_p(att_src2, att_dst2)
    P3 = _build_p(att_src3, att_dst3)

    sc12 = _sc12_build()
    sc3 = _sc3_build()

    *hs, ast, adt = _tc1(x_pad, W1, P1)
    *ogs, den, _unused = sc12(src, dst, ast, adt, *hs)
    *hs, ast, adt = _tc_mid(tuple(ogs), den, b1.reshape(1, 512), W2, P2, 512)
    *ogs, den, _unused = sc12(src, dst, ast, adt, *hs)
    ht, ast, adt = _tc_mid(tuple(ogs), den, b2.reshape(1, 512), W3, P3, 64)
    op, dp = sc3(src, dst, ast, adt, ht)
    return _tc_final(op[0], op[1], dp[0], dp[1], b3.reshape(1, 64))


# speedup vs baseline: 1.0033x; 1.0033x over previous
import functools

import jax
import jax.numpy as jnp
from jax import lax
from jax.experimental import pallas as pl
from jax.experimental.pallas import tpu as pltpu
from jax.experimental.pallas import tpu_sc as plsc

N = 10000
N_PAD = 10240
TRASH = N
E_TOT = 160000 + N
E_PAD = 172032
CHUNK = 128
ROWS_PER_TEC = N_PAD // 16
F32 = jnp.float32

_mesh = lambda: plsc.VectorSubcoreMesh(core_axis_name="c", subcore_axis_name="s")


NG = 8


def _split_cols(hb):
    return tuple(hb[:, g * 64:(g + 1) * 64] for g in range(NG))


def _tc1_body(x_ref, w_ref, p_ref, *outs):
    hb = jnp.dot(x_ref[...], w_ref[...], preferred_element_type=F32)
    ab = jnp.dot(hb, p_ref[...], preferred_element_type=F32)
    hs = _split_cols(hb)
    for g in range(NG):
        outs[g][...] = hs[g]
    z = jnp.zeros((hb.shape[0], 8), F32)
    outs[NG][...] = jnp.concatenate([ab[:, 0:8], z], axis=1)
    outs[NG + 1][...] = jnp.concatenate([ab[:, 8:16], z], axis=1)


def _tc1(x_pad, W1, P1, bn=256):
    grid = (N_PAD // bn,)
    fs = jax.ShapeDtypeStruct
    return pl.pallas_call(
        _tc1_body,
        grid=grid,
        in_specs=[pl.BlockSpec((bn, 256), lambda i: (i, 0)),
                  pl.BlockSpec((256, 512), lambda i: (0, 0)),
                  pl.BlockSpec((512, 16), lambda i: (0, 0))],
        out_specs=[pl.BlockSpec((bn, 64), lambda i: (i, 0))] * NG
                  + [pl.BlockSpec((bn, 16), lambda i: (i, 0))] * 2,
        out_shape=[fs((N_PAD, 64), F32)] * NG + [fs((N_PAD, 16), F32)] * 2,
    )(x_pad, W1, P1)


def _tc_mid_body(*refs):
    og = refs[0:NG]
    den, b_ref, w_ref, p_ref = refs[NG:NG + 4]
    outs = refs[NG + 4:]
    bn = og[0].shape[0]
    d = den[...]
    cols = [og[g][...] / (d[:, g:g + 1] + 1e-16) for g in range(NG)]
    sb = jnp.concatenate(cols, axis=1) + b_ref[...]
    sb = jnp.where(sb > 0, sb, jnp.exp(sb) - 1.0)
    hb = jnp.dot(sb, w_ref[...], preferred_element_type=F32)
    ab = jnp.dot(hb, p_ref[...], preferred_element_type=F32)
    if len(outs) == NG + 2:
        hs = _split_cols(hb)
        for g in range(NG):
            outs[g][...] = hs[g]
        z = jnp.zeros((bn, 8), F32)
        outs[NG][...] = jnp.concatenate([ab[:, 0:8], z], axis=1)
        outs[NG + 1][...] = jnp.concatenate([ab[:, 8:16], z], axis=1)
    else:
        h3_, ast, adt = outs
        h3_[...] = hb
        z15 = jnp.zeros((bn, 15), F32)
        ast[...] = jnp.concatenate([ab[:, 0:1], z15], axis=1)
        adt[...] = jnp.concatenate([ab[:, 8:9], z15], axis=1)


def _tc_mid(og, den, b, W, P, n_out, bn=256):
    grid = (N_PAD // bn,)
    fs = jax.ShapeDtypeStruct
    m = W.shape[1]
    if n_out == 512:
        out_specs = ([pl.BlockSpec((bn, 64), lambda i: (i, 0))] * NG
                     + [pl.BlockSpec((bn, 16), lambda i: (i, 0))] * 2)
        out_shape = [fs((N_PAD, 64), F32)] * NG + [fs((N_PAD, 16), F32)] * 2
    else:
        out_specs = ([pl.BlockSpec((bn, 64), lambda i: (i, 0))]
                     + [pl.BlockSpec((bn, 16), lambda i: (i, 0))] * 2)
        out_shape = [fs((N_PAD, 64), F32)] + [fs((N_PAD, 16), F32)] * 2
    return pl.pallas_call(
        _tc_mid_body,
        grid=grid,
        in_specs=[pl.BlockSpec((bn, 64), lambda i: (i, 0))] * NG
                 + [pl.BlockSpec((bn, 16), lambda i: (i, 0)),
                    pl.BlockSpec((1, 512), lambda i: (0, 0)),
                    pl.BlockSpec((512, m), lambda i: (0, 0)),
                    pl.BlockSpec((m, 16), lambda i: (0, 0))],
        out_specs=out_specs,
        out_shape=out_shape,
    )(*og, den, b, W, P)


def _tc_final_body(p0, p1, d0, d1, b_ref, out_ref):
    den = d0[...][:, 0:1] + d1[...][:, 0:1] + 1e-16
    out_ref[...] = (p0[...] + p1[...]) / den + b_ref[...]


def _tc_final(p0, p1, d0, d1, b3, bn=400):
    grid = (N // bn,)
    return pl.pallas_call(
        _tc_final_body,
        grid=grid,
        in_specs=[pl.BlockSpec((bn, 64), lambda i: (i, 0))] * 2
                 + [pl.BlockSpec((bn, 16), lambda i: (i, 0))] * 2
                 + [pl.BlockSpec((1, 64), lambda i: (0, 0))],
        out_specs=pl.BlockSpec((bn, 64), lambda i: (i, 0)),
        out_shape=jax.ShapeDtypeStruct((N, 64), F32),
    )(p0, p1, d0, d1, b3)


_ZERO16 = None


def _zero_rows(buf, width):
    z = jnp.zeros((16,), F32)

    def body(r, _):
        for j in range(width // 16):
            buf[r, pl.ds(j * 16, 16)] = z
        return 0

    lax.fori_loop(0, CHUNK, body, 0, unroll=4)


def _zero_shared(acc, buf, s, width):
    for k in range(ROWS_PER_TEC // CHUNK):
        pltpu.sync_copy(buf, acc.at[pl.ds(s * ROWS_PER_TEC + k * CHUNK, CHUNK)])


def _alpha_chunk(sbuf, dbuf, abuf):
    def alpha_body(e, _):
        av = sbuf[e, :] + dbuf[e, :]
        av = jnp.maximum(av, 0.2 * av)
        abuf[e, :] = jnp.exp(av)
        return 0

    lax.fori_loop(0, CHUNK, alpha_body, 0, unroll=4)


def _edge_pass(s_row0, n_chunks, sidx_all, didx_all, hg,
               sbuf, dbuf, abuf, hbuf, acc, dacc,
               sa, sd, sh, ssc, ssd, sal,
               a_col, width, add_denom,
               ast_hbm=None, adt_hbm=None,
               alpha_out=None, alpha_in=None, row_base=None):
    nvec = width // 16
    compute_alpha = alpha_in is None

    def issue_gathers(ch, p):
        if compute_alpha:
            pltpu.async_copy(ast_hbm.at[sidx_all.at[s_row0 + ch]], sbuf[p], sa[p])
            pltpu.async_copy(adt_hbm.at[didx_all.at[s_row0 + ch]], dbuf[p], sd[p])
        else:
            pltpu.async_copy(alpha_in.at[pl.ds(row_base + ch * CHUNK, CHUNK)],
                             abuf[p], sa[p])
        pltpu.async_copy(hg.at[sidx_all.at[s_row0 + ch]], hbuf[p], sh[p])

    def wait_scatters(p):
        pltpu.make_async_copy(hbuf[p], acc.at[pl.ds(0, CHUNK)], ssc[p]).wait()
        if add_denom:
            pltpu.make_async_copy(abuf[p], dacc.at[pl.ds(0, CHUNK)], ssd[p]).wait()
        if alpha_out is not None:
            pltpu.make_async_copy(abuf[p], alpha_out.at[pl.ds(0, CHUNK)],
                                  sal[p]).wait()

    issue_gathers(0, 0)

    def loop_body(tt, _):
        for b in (0, 1, 2):
            ch = 3 * tt + b
            p, q = b, (b + 1) % 3

            @pl.when(ch >= 2)
            def _():
                wait_scatters(q)

            @pl.when(ch + 1 < n_chunks)
            def _():
                issue_gathers(ch + 1, q)

            if compute_alpha:
                pltpu.make_async_copy(ast_hbm.at[pl.ds(0, CHUNK)], sbuf[p],
                                      sa[p]).wait()
                pltpu.make_async_copy(adt_hbm.at[pl.ds(0, CHUNK)], dbuf[p],
                                      sd[p]).wait()
                _alpha_chunk(sbuf[p], dbuf[p], abuf[p])
                if alpha_out is not None:
                    pltpu.async_copy(abuf[p],
                                     alpha_out.at[pl.ds(row_base + ch * CHUNK,
                                                        CHUNK)], sal[p])
                if add_denom:
                    pltpu.async_copy(abuf[p], dacc.at[didx_all.at[s_row0 + ch]],
                                     ssd[p], add=True)
            else:
                pltpu.make_async_copy(alpha_in.at[pl.ds(0, CHUNK)], abuf[p],
                                      sa[p]).wait()
            pltpu.make_async_copy(hg.at[pl.ds(0, CHUNK)], hbuf[p], sh[p]).wait()

            def scale_body(e, _):
                av = abuf[p][e, :]
                for j in range(nvec):
                    aa = av[a_col + (j * 16) // 64]
                    hbuf[p][e, pl.ds(j * 16, 16)] = hbuf[p][e, pl.ds(j * 16, 16)] * aa
                return 0

            lax.fori_loop(0, CHUNK, scale_body, 0, unroll=4)
            pltpu.async_copy(hbuf[p], acc.at[didx_all.at[s_row0 + ch]],
                             ssc[p], add=True)
        return 0

    lax.fori_loop(0, n_chunks // 3, loop_body, 0)
    wait_scatters(1)
    wait_scatters(2)


def _sc12_build():
    fs = jax.ShapeDtypeStruct
    per_tec = E_PAD // 16
    n_chunks = per_tec // CHUNK

    @functools.partial(
        pl.kernel,
        out_type=[fs((N_PAD, 64), F32)] * NG + [fs((N_PAD, 16), F32),
                                                fs((2 * E_PAD, 16), F32)],
        mesh=_mesh(),
        compiler_params=pltpu.CompilerParams(use_tc_tiling_on_sc=False),
        scratch_types=[
            pltpu.VMEM((n_chunks, CHUNK), jnp.int32),
            pltpu.VMEM((n_chunks, CHUNK), jnp.int32),
            [pltpu.VMEM((CHUNK, 16), F32)] * 3,
            [pltpu.VMEM((CHUNK, 16), F32)] * 3,
            [pltpu.VMEM((CHUNK, 16), F32)] * 3,
            [pltpu.VMEM((CHUNK, 64), F32)] * 3,
            pltpu.VMEM_SHARED((N_PAD, 64), F32),
            pltpu.VMEM_SHARED((N_PAD, 16), F32),
            [pltpu.SemaphoreType.DMA] * 3,
            [pltpu.SemaphoreType.DMA] * 3,
            [pltpu.SemaphoreType.DMA] * 3,
            [pltpu.SemaphoreType.DMA] * 3,
            [pltpu.SemaphoreType.DMA] * 3,
            [pltpu.SemaphoreType.DMA] * 3,
        ],
    )
    def sc12(src_hbm, dst_hbm, ast_hbm, adt_hbm, *rest):
        hgs = rest[0:NG]
        ogs = rest[NG:2 * NG]
        den_out = rest[2 * NG]
        alpha_hbm = rest[2 * NG + 1]
        (sidx_all, didx_all, sbuf, dbuf, abuf, hbuf, acc, dacc,
         sa, sd, sh, ssc, ssd, sal) = rest[2 * NG + 2:]
        c = lax.axis_index("c")
        s = lax.axis_index("s")

        pltpu.sync_copy(src_hbm.at[pl.ds(s * n_chunks, n_chunks)], sidx_all)
        pltpu.sync_copy(dst_hbm.at[pl.ds(s * n_chunks, n_chunks)], didx_all)

        _zero_rows(hbuf[0], 64)
        _zero_shared(acc, hbuf[0], s, 64)
        _zero_rows(sbuf[0], 16)
        _zero_shared(dacc, sbuf[0], s, 16)
        plsc.subcore_barrier()

        def dump(og):
            pltpu.sync_copy(acc.at[pl.ds(s * ROWS_PER_TEC, ROWS_PER_TEC)],
                            og.at[pl.ds(s * ROWS_PER_TEC, ROWS_PER_TEC)])

        def do_groups(c_val):
            row_base = c_val * E_PAD + s * per_tec
            for k in range(4):
                g = 4 * c_val + k
                add_denom = (c_val == 0 and k == 0)
                if k > 0:
                    _zero_rows(hbuf[0], 64)
                    _zero_shared(acc, hbuf[0], s, 64)
                    plsc.subcore_barrier()
                if k == 0:
                    _edge_pass(0, n_chunks, sidx_all, didx_all,
                               hgs[g], sbuf, dbuf, abuf, hbuf, acc, dacc,
                               sa, sd, sh, ssc, ssd, sal, g, 64, add_denom,
                               ast_hbm=ast_hbm, adt_hbm=adt_hbm,
                               alpha_out=alpha_hbm, row_base=row_base)
                else:
                    _edge_pass(0, n_chunks, sidx_all, didx_all,
                               hgs[g], sbuf, dbuf, abuf, hbuf, acc, dacc,
                               sa, sd, sh, ssc, ssd, sal, g, 64, False,
                               alpha_in=alpha_hbm, row_base=row_base)
                plsc.subcore_barrier()
                dump(ogs[g])
                if add_denom:
                    pltpu.sync_copy(
                        dacc.at[pl.ds(s * ROWS_PER_TEC, ROWS_PER_TEC)],
                        den_out.at[pl.ds(s * ROWS_PER_TEC, ROWS_PER_TEC)])

        @pl.when(c == 0)
        def _():
            do_groups(0)

        @pl.when(c == 1)
        def _():
            do_groups(1)

    return sc12


def _sc3_build():
    fs = jax.ShapeDtypeStruct
    per_tec = E_PAD // 32

    n_chunks = per_tec // CHUNK

    @functools.partial(
        pl.kernel,
        out_type=[fs((2, N_PAD, 64), F32), fs((2, N_PAD, 16), F32)],
        mesh=_mesh(),
        compiler_params=pltpu.CompilerParams(use_tc_tiling_on_sc=False),
        scratch_types=[
            pltpu.VMEM((n_chunks, CHUNK), jnp.int32),
            pltpu.VMEM((n_chunks, CHUNK), jnp.int32),
            [pltpu.VMEM((CHUNK, 16), F32)] * 3,
            [pltpu.VMEM((CHUNK, 16), F32)] * 3,
            [pltpu.VMEM((CHUNK, 16), F32)] * 3,
            [pltpu.VMEM((CHUNK, 64), F32)] * 3,
            pltpu.VMEM_SHARED((N_PAD, 64), F32),
            pltpu.VMEM_SHARED((N_PAD, 16), F32),
            [pltpu.SemaphoreType.DMA] * 3,
            [pltpu.SemaphoreType.DMA] * 3,
            [pltpu.SemaphoreType.DMA] * 3,
            [pltpu.SemaphoreType.DMA] * 3,
            [pltpu.SemaphoreType.DMA] * 3,
        ],
    )
    def sc3(src_hbm, dst_hbm, ast_hbm, adt_hbm, h3_hbm,
            op, dp,
            sidx_all, didx_all, sbuf, dbuf, abuf, hbuf, acc, dacc,
            sa, sd, sh, ssc, ssd):
        c = lax.axis_index("c")
        s = lax.axis_index("s")
        wid = c * 16 + s

        pltpu.sync_copy(src_hbm.at[pl.ds(wid * n_chunks, n_chunks)], sidx_all)
        pltpu.sync_copy(dst_hbm.at[pl.ds(wid * n_chunks, n_chunks)], didx_all)

        _zero_rows(hbuf[0], 64)
        _zero_shared(acc, hbuf[0], s, 64)
        _zero_rows(sbuf[0], 16)
        _zero_shared(dacc, sbuf[0], s, 16)
        plsc.subcore_barrier()

        _edge_pass(0, n_chunks, sidx_all, didx_all, h3_hbm,
                   sbuf, dbuf, abuf, hbuf, acc, dacc,
                   sa, sd, sh, ssc, ssd, None, 0, 64, True,
                   ast_hbm=ast_hbm, adt_hbm=adt_hbm)
        plsc.subcore_barrier()
        pltpu.sync_copy(acc.at[pl.ds(s * ROWS_PER_TEC, ROWS_PER_TEC)],
                        op.at[c, pl.ds(s * ROWS_PER_TEC, ROWS_PER_TEC)])
        pltpu.sync_copy(dacc.at[pl.ds(s * ROWS_PER_TEC, ROWS_PER_TEC)],
                        dp.at[c, pl.ds(s * ROWS_PER_TEC, ROWS_PER_TEC)])

    return sc3


def _bp(att):
    H, C = att.shape
    eye = jnp.eye(H, dtype=att.dtype)
    return (att[:, :, None] * eye[:, None, :]).reshape(H * C, H)


def _build_p(att_s, att_d):
    H = att_s.shape[0]
    ps, pd = _bp(att_s), _bp(att_d)
    z = jnp.zeros((ps.shape[0], 8 - H), att_s.dtype)
    return jnp.concatenate([ps, z, pd, z], axis=1)


def kernel(x, edge_index, W1, att_src1, att_dst1, b1,
           W2, att_src2, att_dst2, b2,
           W3, att_src3, att_dst3, b3):
    ei = edge_index.astype(jnp.int32)
    loop = jnp.arange(N, dtype=jnp.int32)
    pad_n = E_PAD - E_TOT
    src = jnp.concatenate([ei[0], loop, jnp.zeros((pad_n,), jnp.int32)])
    dst = jnp.concatenate([ei[1], loop, jnp.full((pad_n,), TRASH, jnp.int32)])
    src = src.reshape(E_PAD // CHUNK, CHUNK)
    dst = dst.reshape(E_PAD // CHUNK, CHUNK)
    x_pad = jnp.pad(x, ((0, N_PAD - N), (0, 0)))

    P1 = _build_p(att_src1, att_dst1)
    P2 = _build_p(att_src2, att_dst2)
    P3 = _build_p(att_src3, att_dst3)

    sc12 = _sc12_build()
    sc3 = _sc3_build()

    *hs, ast, adt = _tc1(x_pad, W1, P1)
    *ogs, den, _unused = sc12(src, dst, ast, adt, *hs)
    *hs, ast, adt = _tc_mid(tuple(ogs), den, b1.reshape(1, 512), W2, P2, 512)
    *ogs, den, _unused = sc12(src, dst, ast, adt, *hs)
    ht, ast, adt = _tc_mid(tuple(ogs), den, b2.reshape(1, 512), W3, P3, 64)
    op, dp = sc3(src, dst, ast, adt, ht)
    return _tc_final(op[0], op[1], dp[0], dp[1], b3.reshape(1, 64))
